# Initial kernel scaffold; baseline (speedup 1.0000x reference)
import jax
import jax.numpy as jnp
from jax.experimental import pallas as pl


def kernel(edge_index, num_nodes, W1, b1, W2, b2, W3, b3, W4, b4, Wc, bc):
    raise NotImplementedError("write your pallas kernel here")


# speedup vs baseline: 41.3826x; 41.3826x over previous
import functools

import jax
import jax.numpy as jnp
from jax import lax
from jax.experimental import pallas as pl
from jax.experimental.pallas import tpu as pltpu
from jax.experimental.pallas import tpu_sc as plsc

N = 100000
E = 3200000
NPAD = 100096
SENT = N
NC = 2
NS = 16
NTILES = NC * NS
RPT = 784
EROWS = NTILES * RPT
EPAD = EROWS * 128
BR = 56
NBIG = RPT // BR
NB = 7
NGRP = BR // NB
OROWS = NPAD // NS

_mesh = plsc.VectorSubcoreMesh(
    core_axis_name="c", subcore_axis_name="s", num_cores=NC, num_subcores=NS)


def _make_prop(w):
  @functools.partial(
      pl.kernel,
      out_type=jax.ShapeDtypeStruct((NC, NPAD, w), jnp.float32),
      mesh=_mesh,
      compiler_params=pltpu.CompilerParams(use_tc_tiling_on_sc=False),
      scratch_types=[
          pltpu.VMEM((2, BR, 128), jnp.int32),
          pltpu.VMEM((NB, 128, w), jnp.float32),
          pltpu.VMEM_SHARED((NPAD, w), jnp.float32),
          pltpu.SemaphoreType.DMA((NB,)),
          pltpu.SemaphoreType.DMA((NB,)),
      ],
  )
  def prop(x_hbm, src_hbm, dst_hbm, zeros_hbm, out_hbm,
           idx_v, rows_v, acc_sh, gsem, ssem):
    cid = lax.axis_index("c")
    sid = lax.axis_index("s")
    wid = cid * NS + sid
    pltpu.sync_copy(zeros_hbm.at[pl.ds(sid * OROWS, OROWS)],
                    acc_sh.at[pl.ds(sid * OROWS, OROWS)])
    plsc.subcore_barrier()
    row0 = wid * RPT

    def big_body(big, carry):
      r0 = row0 + big * BR
      pltpu.sync_copy(src_hbm.at[pl.ds(r0, BR)], idx_v.at[0])
      pltpu.sync_copy(dst_hbm.at[pl.ds(r0, BR)], idx_v.at[1])

      def grp_body(g, c):
        gd = []
        for b in range(NB):
          gd.append(pltpu.async_copy(
              x_hbm.at[idx_v.at[0, g * NB + b]], rows_v.at[b], gsem.at[b]))
        sd = []
        for b in range(NB):
          gd[b].wait()
          sd.append(pltpu.async_copy(
              rows_v.at[b], acc_sh.at[idx_v.at[1, g * NB + b]],
              ssem.at[b], add=True))
        for b in range(NB):
          sd[b].wait()
        return c

      return lax.fori_loop(0, NGRP, grp_body, carry)

    lax.fori_loop(0, NBIG, big_body, 0)
    plsc.subcore_barrier()
    pltpu.sync_copy(acc_sh.at[pl.ds(sid * OROWS, OROWS)],
                    out_hbm.at[cid, pl.ds(sid * OROWS, OROWS)])

  return prop


_prop16 = _make_prop(16)
_prop8 = _make_prop(8)


def _mask_ids(x):
  r = lax.broadcasted_iota(jnp.int32, x.shape, x.ndim - 2)
  l = lax.broadcasted_iota(jnp.int32, x.shape, x.ndim - 1)
  return jnp.where(r * 128 + l < N, x, 0.0)


NR = NPAD // 128


def _stage0(din_ref, dout_ref, x1_ref, nd_ref, ns_ref):
  in_deg = din_ref[0, 0] + din_ref[1, 0]
  out_deg = dout_ref[0, 0] + dout_ref[1, 0]
  ns = lax.rsqrt(jnp.maximum(out_deg, 1.0))
  nd = lax.rsqrt(jnp.maximum(in_deg, 1.0))
  x1_ref[...] = _mask_ids(in_deg * ns)
  nd_ref[...] = nd
  ns_ref[...] = ns


def _stage1(mp_ref, nd_ref, ns_ref, w1_ref, b1_ref, x2_ref):
  m = (mp_ref[0, 0] + mp_ref[1, 0]) * nd_ref[...]
  w1 = jnp.reshape(w1_ref[...], (16, 1, 1))
  b1 = jnp.reshape(b1_ref[...], (16, 1, 1))
  h = jnp.maximum(m[None] * w1 + b1, 0.0)
  x2_ref[...] = _mask_ids(h * ns_ref[...][None])


def _matT(w_ref, x):
  win, wout = w_ref.shape
  flat = jnp.reshape(x, (win, NR * 128))
  out = lax.dot_general(w_ref[...], flat, (((0,), (0,)), ((), ())),
                        preferred_element_type=jnp.float32)
  return jnp.reshape(out, (wout, NR, 128))


def _stage2(mp_ref, nd_ref, ns_ref, w2_ref, b2_ref, w3_ref, x3_ref):
  m = (mp_ref[0] + mp_ref[1]) * nd_ref[...][None]
  b2 = jnp.reshape(b2_ref[...], (16, 1, 1))
  h = jnp.maximum(_matT(w2_ref, m) + b2, 0.0)
  x3_ref[...] = _mask_ids(_matT(w3_ref, h) * ns_ref[...][None])


def _stage3(mp_ref, nd_ref, ns_ref, b3_ref, w4_ref, x4_ref):
  m = (mp_ref[0] + mp_ref[1]) * nd_ref[...][None]
  b3 = jnp.reshape(b3_ref[...], (8, 1, 1))
  h = jnp.maximum(m + b3, 0.0)
  x4_ref[...] = _mask_ids(_matT(w4_ref, h) * ns_ref[...][None])


def _stage4(mp_ref, nd_ref, b4_ref, wc_ref, bc_ref, out_ref):
  m = (mp_ref[0] + mp_ref[1])[:4] * nd_ref[...][None]
  b4 = jnp.reshape(b4_ref[...], (4, 1, 1))
  h = _mask_ids(jnp.maximum(m + b4, 0.0))
  g = jnp.sum(jnp.reshape(h, (4, NR * 128)), axis=1, keepdims=True)
  g = jnp.transpose(g) * (1.0 / N)
  out_ref[...] = (
      jnp.dot(g, wc_ref[...], preferred_element_type=jnp.float32)
      + bc_ref[...])


def _tc(body, out_shape, *args):
  return pl.pallas_call(body, out_shape=out_shape)(*args)


def _f32(*shape):
  return jax.ShapeDtypeStruct(shape, jnp.float32)


def _to_sc(xT, w):
  return jnp.transpose(jnp.reshape(xT, (w, NPAD)))


def _to_tc(m, w):
  return jnp.reshape(jnp.transpose(m, (0, 2, 1)), (NC, w, NR, 128))


def kernel(edge_index, num_nodes, W1, b1, W2, b2, W3, b3, W4, b4, Wc, bc):
  del num_nodes
  pad = jnp.full((EPAD - E,), SENT, jnp.int32)
  src2 = jnp.concatenate([edge_index[0], pad]).reshape(EROWS, 128)
  dst2 = jnp.concatenate([edge_index[1], pad]).reshape(EROWS, 128)
  ones8 = jnp.ones((NPAD, 8), jnp.float32)
  z8 = jnp.zeros((NPAD, 8), jnp.float32)
  z16 = jnp.zeros((NPAD, 16), jnp.float32)

  din = _prop8(ones8, src2, dst2, z8)
  dout = _prop8(ones8, dst2, src2, z8)
  x1, nd, ns = _tc(
      _stage0, (_f32(NR, 128), _f32(NR, 128), _f32(NR, 128)),
      _to_tc(din, 8), _to_tc(dout, 8))
  x1_8 = jnp.concatenate(
      [jnp.reshape(x1, (NPAD, 1)), jnp.zeros((NPAD, 7), jnp.float32)], axis=1)
  m1 = _prop8(x1_8, src2, dst2, z8)
  x2 = _tc(_stage1, _f32(16, NR, 128), _to_tc(m1, 8), nd, ns,
           W1, b1.reshape(1, 16))
  m2 = _prop16(_to_sc(x2, 16), src2, dst2, z16)
  x3 = _tc(_stage2, _f32(8, NR, 128), _to_tc(m2, 16), nd, ns,
           W2, b2.reshape(1, 16), W3)
  m3 = _prop8(_to_sc(x3, 8), src2, dst2, z8)
  w4p = jnp.concatenate([W4, jnp.zeros((8, 4), jnp.float32)], axis=1)
  x4 = _tc(_stage3, _f32(8, NR, 128), _to_tc(m3, 8), nd, ns,
           b3.reshape(1, 8), w4p)
  m4 = _prop8(_to_sc(x4, 8), src2, dst2, z8)
  out = _tc(_stage4, _f32(1, 10), _to_tc(m4, 8), nd,
            b4.reshape(1, 4), Wc, bc.reshape(1, 10))
  return out


# speedup vs baseline: 46.3682x; 1.1205x over previous
import functools

import jax
import jax.numpy as jnp
from jax import lax
from jax.experimental import pallas as pl
from jax.experimental.pallas import tpu as pltpu
from jax.experimental.pallas import tpu_sc as plsc

N = 100000
E = 3200000
NPAD = 100096
SENT = N
NC = 2
NS = 16
NTILES = NC * NS
RPT = 784
EROWS = NTILES * RPT
EPAD = EROWS * 128
BR = 56
NBIG = RPT // BR
NB = 7
NGRP = BR // NB
OROWS = NPAD // NS

_mesh = plsc.VectorSubcoreMesh(
    core_axis_name="c", subcore_axis_name="s", num_cores=NC, num_subcores=NS)


def _make_prop(w, br, nb):
  nbig = RPT // br
  ngrp = br // nb

  @functools.partial(
      pl.kernel,
      out_type=jax.ShapeDtypeStruct((NC, NPAD, w), jnp.float32),
      mesh=_mesh,
      compiler_params=pltpu.CompilerParams(use_tc_tiling_on_sc=False),
      scratch_types=[
          pltpu.VMEM((2, br, 128), jnp.int32),
          pltpu.VMEM((nb, 128, w), jnp.float32),
          pltpu.VMEM_SHARED((NPAD, w), jnp.float32),
          pltpu.SemaphoreType.DMA((nb,)),
          pltpu.SemaphoreType.DMA((nb,)),
      ],
  )
  def prop(x_hbm, src_hbm, dst_hbm, zeros_hbm, out_hbm,
           idx_v, rows_v, acc_sh, gsem, ssem):
    cid = lax.axis_index("c")
    sid = lax.axis_index("s")
    wid = cid * NS + sid
    pltpu.sync_copy(zeros_hbm.at[pl.ds(sid * OROWS, OROWS)],
                    acc_sh.at[pl.ds(sid * OROWS, OROWS)])
    plsc.subcore_barrier()
    row0 = wid * RPT

    def big_body(big, carry):
      r0 = row0 + big * br
      pltpu.sync_copy(src_hbm.at[pl.ds(r0, br)], idx_v.at[0])
      pltpu.sync_copy(dst_hbm.at[pl.ds(r0, br)], idx_v.at[1])

      def grp_body(g, c):
        first = jnp.logical_and(big == 0, g == 0)
        for b in range(nb):
          @pl.when(jnp.logical_not(first))
          def _():
            pltpu.make_async_copy(rows_v.at[b],
                                  acc_sh.at[idx_v.at[1, 0]],
                                  ssem.at[b]).wait()
          pltpu.async_copy(x_hbm.at[idx_v.at[0, g * nb + b]],
                           rows_v.at[b], gsem.at[b])
        for b in range(nb):
          pltpu.make_async_copy(x_hbm.at[idx_v.at[0, 0]],
                                rows_v.at[b], gsem.at[b]).wait()
          pltpu.async_copy(rows_v.at[b],
                           acc_sh.at[idx_v.at[1, g * nb + b]],
                           ssem.at[b], add=True)
        return c

      return lax.fori_loop(0, ngrp, grp_body, carry)

    lax.fori_loop(0, nbig, big_body, 0)
    for b in range(nb):
      pltpu.make_async_copy(rows_v.at[b], acc_sh.at[idx_v.at[1, 0]],
                            ssem.at[b]).wait()
    plsc.subcore_barrier()
    pltpu.sync_copy(acc_sh.at[pl.ds(sid * OROWS, OROWS)],
                    out_hbm.at[cid, pl.ds(sid * OROWS, OROWS)])

  return prop


@functools.partial(
    pl.kernel,
    out_type=jax.ShapeDtypeStruct((NC, 2, NPAD, 8), jnp.float32),
    mesh=_mesh,
    compiler_params=pltpu.CompilerParams(use_tc_tiling_on_sc=False),
    scratch_types=[
        pltpu.VMEM((2, BR, 128), jnp.int32),
        pltpu.VMEM((128, 8), jnp.float32),
        pltpu.VMEM_SHARED((NPAD, 8), jnp.float32),
        pltpu.VMEM_SHARED((NPAD, 8), jnp.float32),
        pltpu.SemaphoreType.DMA((NB,)),
        pltpu.SemaphoreType.DMA((NB,)),
    ],
)
def _degrees(src_hbm, dst_hbm, ones_hbm, zeros_hbm, out_hbm,
             idx_v, ones_v, accin_sh, accout_sh, isem, osem):
  cid = lax.axis_index("c")
  sid = lax.axis_index("s")
  wid = cid * NS + sid
  pltpu.sync_copy(ones_hbm, ones_v)
  pltpu.sync_copy(zeros_hbm.at[pl.ds(sid * OROWS, OROWS)],
                  accin_sh.at[pl.ds(sid * OROWS, OROWS)])
  pltpu.sync_copy(zeros_hbm.at[pl.ds(sid * OROWS, OROWS)],
                  accout_sh.at[pl.ds(sid * OROWS, OROWS)])
  plsc.subcore_barrier()
  row0 = wid * RPT

  def big_body(big, carry):
    r0 = row0 + big * BR
    pltpu.sync_copy(src_hbm.at[pl.ds(r0, BR)], idx_v.at[0])
    pltpu.sync_copy(dst_hbm.at[pl.ds(r0, BR)], idx_v.at[1])

    def grp_body(g, c):
      for b in range(NB):
        r = g * NB + b
        pltpu.async_copy(ones_v, accin_sh.at[idx_v.at[1, r]],
                         isem.at[b], add=True)
        pltpu.async_copy(ones_v, accout_sh.at[idx_v.at[0, r]],
                         osem.at[b], add=True)
      for b in range(NB):
        pltpu.make_async_copy(ones_v, accin_sh.at[idx_v.at[1, 0]],
                              isem.at[b]).wait()
        pltpu.make_async_copy(ones_v, accout_sh.at[idx_v.at[0, 0]],
                              osem.at[b]).wait()
      return c

    return lax.fori_loop(0, NGRP, grp_body, carry)

  lax.fori_loop(0, NBIG, big_body, 0)
  plsc.subcore_barrier()
  pltpu.sync_copy(accin_sh.at[pl.ds(sid * OROWS, OROWS)],
                  out_hbm.at[cid, 0, pl.ds(sid * OROWS, OROWS)])
  pltpu.sync_copy(accout_sh.at[pl.ds(sid * OROWS, OROWS)],
                  out_hbm.at[cid, 1, pl.ds(sid * OROWS, OROWS)])


_prop16 = _make_prop(16, 56, 7)
_prop8 = _make_prop(8, 56, 14)


def _mask_ids(x):
  r = lax.broadcasted_iota(jnp.int32, x.shape, x.ndim - 2)
  l = lax.broadcasted_iota(jnp.int32, x.shape, x.ndim - 1)
  return jnp.where(r * 128 + l < N, x, 0.0)


NR = NPAD // 128


def _stage0(din_ref, dout_ref, x1_ref, nd_ref, ns_ref):
  in_deg = din_ref[0, 0] + din_ref[1, 0]
  out_deg = dout_ref[0, 0] + dout_ref[1, 0]
  ns = lax.rsqrt(jnp.maximum(out_deg, 1.0))
  nd = lax.rsqrt(jnp.maximum(in_deg, 1.0))
  x1_ref[...] = _mask_ids(in_deg * ns)
  nd_ref[...] = nd
  ns_ref[...] = ns


def _stage1(mp_ref, nd_ref, ns_ref, w1_ref, b1_ref, x2_ref):
  m = (mp_ref[0, 0] + mp_ref[1, 0]) * nd_ref[...]
  w1 = jnp.reshape(w1_ref[...], (16, 1, 1))
  b1 = jnp.reshape(b1_ref[...], (16, 1, 1))
  h = jnp.maximum(m[None] * w1 + b1, 0.0)
  x2_ref[...] = _mask_ids(h * ns_ref[...][None])


def _matT(w_ref, x):
  win, wout = w_ref.shape
  flat = jnp.reshape(x, (win, NR * 128))
  out = lax.dot_general(w_ref[...], flat, (((0,), (0,)), ((), ())),
                        preferred_element_type=jnp.float32)
  return jnp.reshape(out, (wout, NR, 128))


def _stage2(mp_ref, nd_ref, ns_ref, w2_ref, b2_ref, w3_ref, x3_ref):
  m = (mp_ref[0] + mp_ref[1]) * nd_ref[...][None]
  b2 = jnp.reshape(b2_ref[...], (16, 1, 1))
  h = jnp.maximum(_matT(w2_ref, m) + b2, 0.0)
  x3_ref[...] = _mask_ids(_matT(w3_ref, h) * ns_ref[...][None])


def _stage3(mp_ref, nd_ref, ns_ref, b3_ref, w4_ref, x4_ref):
  m = (mp_ref[0] + mp_ref[1]) * nd_ref[...][None]
  b3 = jnp.reshape(b3_ref[...], (8, 1, 1))
  h = jnp.maximum(m + b3, 0.0)
  x4_ref[...] = _mask_ids(_matT(w4_ref, h) * ns_ref[...][None])


def _stage4(mp_ref, nd_ref, b4_ref, wc_ref, bc_ref, out_ref):
  m = (mp_ref[0] + mp_ref[1])[:4] * nd_ref[...][None]
  b4 = jnp.reshape(b4_ref[...], (4, 1, 1))
  h = _mask_ids(jnp.maximum(m + b4, 0.0))
  g = jnp.sum(jnp.reshape(h, (4, NR * 128)), axis=1, keepdims=True)
  g = jnp.transpose(g) * (1.0 / N)
  out_ref[...] = (
      jnp.dot(g, wc_ref[...], preferred_element_type=jnp.float32)
      + bc_ref[...])


def _tc(body, out_shape, *args):
  return pl.pallas_call(body, out_shape=out_shape)(*args)


def _f32(*shape):
  return jax.ShapeDtypeStruct(shape, jnp.float32)


def _to_sc(xT, w):
  return jnp.transpose(jnp.reshape(xT, (w, NPAD)))


def _to_tc(m, w):
  return jnp.reshape(jnp.transpose(m, (0, 2, 1)), (NC, w, NR, 128))


def kernel(edge_index, num_nodes, W1, b1, W2, b2, W3, b3, W4, b4, Wc, bc):
  del num_nodes
  pad = jnp.full((EPAD - E,), SENT, jnp.int32)
  src2 = jnp.concatenate([edge_index[0], pad]).reshape(EROWS, 128)
  dst2 = jnp.concatenate([edge_index[1], pad]).reshape(EROWS, 128)
  ones8 = jnp.ones((128, 8), jnp.float32)
  z8 = jnp.zeros((NPAD, 8), jnp.float32)
  z16 = jnp.zeros((NPAD, 16), jnp.float32)

  degp = _degrees(src2, dst2, ones8, z8)
  x1, nd, ns = _tc(
      _stage0, (_f32(NR, 128), _f32(NR, 128), _f32(NR, 128)),
      _to_tc(degp[:, 0], 8), _to_tc(degp[:, 1], 8))
  x1_8 = jnp.concatenate(
      [jnp.reshape(x1, (NPAD, 1)), jnp.zeros((NPAD, 7), jnp.float32)], axis=1)
  m1 = _prop8(x1_8, src2, dst2, z8)
  x2 = _tc(_stage1, _f32(16, NR, 128), _to_tc(m1, 8), nd, ns,
           W1, b1.reshape(1, 16))
  m2 = _prop16(_to_sc(x2, 16), src2, dst2, z16)
  x3 = _tc(_stage2, _f32(8, NR, 128), _to_tc(m2, 16), nd, ns,
           W2, b2.reshape(1, 16), W3)
  m3 = _prop8(_to_sc(x3, 8), src2, dst2, z8)
  w4p = jnp.concatenate([W4, jnp.zeros((8, 4), jnp.float32)], axis=1)
  x4 = _tc(_stage3, _f32(8, NR, 128), _to_tc(m3, 8), nd, ns,
           b3.reshape(1, 8), w4p)
  m4 = _prop8(_to_sc(x4, 8), src2, dst2, z8)
  out = _tc(_stage4, _f32(1, 10), _to_tc(m4, 8), nd,
            b4.reshape(1, 4), Wc, bc.reshape(1, 10))
  return out


# speedup vs baseline: 82.1949x; 1.7727x over previous
import functools

import jax
import jax.numpy as jnp
from jax import lax
from jax.experimental import pallas as pl
from jax.experimental.pallas import tpu as pltpu
from jax.experimental.pallas import tpu_sc as plsc

N = 100000
E = 3200000
NPAD = 100096
SENT = N
NC = 2
NS = 16
NTILES = NC * NS
RPT = 784
EROWS = NTILES * RPT
EPAD = EROWS * 128
BR = 56
NBIG = RPT // BR
NB = 7
NGRP = BR // NB
OROWS = NPAD // NS

_mesh = plsc.VectorSubcoreMesh(
    core_axis_name="c", subcore_axis_name="s", num_cores=NC, num_subcores=NS)


def _make_prop(w, br, nb):
  nbig = RPT // br
  ngrp = br // nb

  @functools.partial(
      pl.kernel,
      out_type=jax.ShapeDtypeStruct((NC, NPAD, w), jnp.float32),
      mesh=_mesh,
      compiler_params=pltpu.CompilerParams(use_tc_tiling_on_sc=False),
      scratch_types=[
          pltpu.VMEM((2, br, 128), jnp.int32),
          pltpu.VMEM((nb, 128, w), jnp.float32),
          pltpu.VMEM_SHARED((NPAD, w), jnp.float32),
          pltpu.SemaphoreType.DMA((nb,)),
          pltpu.SemaphoreType.DMA((nb,)),
      ],
  )
  def prop(x_hbm, src_hbm, dst_hbm, zeros_hbm, out_hbm,
           idx_v, rows_v, acc_sh, gsem, ssem):
    cid = lax.axis_index("c")
    sid = lax.axis_index("s")
    wid = cid * NS + sid
    pltpu.sync_copy(zeros_hbm.at[pl.ds(sid * OROWS, OROWS)],
                    acc_sh.at[pl.ds(sid * OROWS, OROWS)])
    plsc.subcore_barrier()
    row0 = wid * RPT

    def big_body(big, carry):
      r0 = row0 + big * br
      pltpu.sync_copy(src_hbm.at[pl.ds(r0, br)], idx_v.at[0])
      pltpu.sync_copy(dst_hbm.at[pl.ds(r0, br)], idx_v.at[1])

      def grp_body(g, c):
        first = jnp.logical_and(big == 0, g == 0)
        for b in range(nb):
          @pl.when(jnp.logical_not(first))
          def _():
            pltpu.make_async_copy(rows_v.at[b],
                                  acc_sh.at[idx_v.at[1, 0]],
                                  ssem.at[b]).wait()
          pltpu.async_copy(x_hbm.at[idx_v.at[0, g * nb + b]],
                           rows_v.at[b], gsem.at[b])
        for b in range(nb):
          pltpu.make_async_copy(x_hbm.at[idx_v.at[0, 0]],
                                rows_v.at[b], gsem.at[b]).wait()
          pltpu.async_copy(rows_v.at[b],
                           acc_sh.at[idx_v.at[1, g * nb + b]],
                           ssem.at[b], add=True)
        return c

      return lax.fori_loop(0, ngrp, grp_body, carry)

    lax.fori_loop(0, nbig, big_body, 0)
    for b in range(nb):
      pltpu.make_async_copy(rows_v.at[b], acc_sh.at[idx_v.at[1, 0]],
                            ssem.at[b]).wait()
    plsc.subcore_barrier()
    pltpu.sync_copy(acc_sh.at[pl.ds(sid * OROWS, OROWS)],
                    out_hbm.at[cid, pl.ds(sid * OROWS, OROWS)])

  return prop


@functools.partial(
    pl.kernel,
    out_type=jax.ShapeDtypeStruct((NC, 2, NPAD, 8), jnp.float32),
    mesh=_mesh,
    compiler_params=pltpu.CompilerParams(use_tc_tiling_on_sc=False),
    scratch_types=[
        pltpu.VMEM((2, BR, 128), jnp.int32),
        pltpu.VMEM((128, 8), jnp.float32),
        pltpu.VMEM_SHARED((NPAD, 8), jnp.float32),
        pltpu.VMEM_SHARED((NPAD, 8), jnp.float32),
        pltpu.SemaphoreType.DMA((NB,)),
        pltpu.SemaphoreType.DMA((NB,)),
    ],
)
def _degrees(src_hbm, dst_hbm, ones_hbm, zeros_hbm, out_hbm,
             idx_v, ones_v, accin_sh, accout_sh, isem, osem):
  cid = lax.axis_index("c")
  sid = lax.axis_index("s")
  wid = cid * NS + sid
  pltpu.sync_copy(ones_hbm, ones_v)
  pltpu.sync_copy(zeros_hbm.at[pl.ds(sid * OROWS, OROWS)],
                  accin_sh.at[pl.ds(sid * OROWS, OROWS)])
  pltpu.sync_copy(zeros_hbm.at[pl.ds(sid * OROWS, OROWS)],
                  accout_sh.at[pl.ds(sid * OROWS, OROWS)])
  plsc.subcore_barrier()
  row0 = wid * RPT

  def big_body(big, carry):
    r0 = row0 + big * BR
    pltpu.sync_copy(src_hbm.at[pl.ds(r0, BR)], idx_v.at[0])
    pltpu.sync_copy(dst_hbm.at[pl.ds(r0, BR)], idx_v.at[1])

    def grp_body(g, c):
      for b in range(NB):
        r = g * NB + b
        pltpu.async_copy(ones_v, accin_sh.at[idx_v.at[1, r]],
                         isem.at[b], add=True)
        pltpu.async_copy(ones_v, accout_sh.at[idx_v.at[0, r]],
                         osem.at[b], add=True)
      for b in range(NB):
        pltpu.make_async_copy(ones_v, accin_sh.at[idx_v.at[1, 0]],
                              isem.at[b]).wait()
        pltpu.make_async_copy(ones_v, accout_sh.at[idx_v.at[0, 0]],
                              osem.at[b]).wait()
      return c

    return lax.fori_loop(0, NGRP, grp_body, carry)

  lax.fori_loop(0, NBIG, big_body, 0)
  plsc.subcore_barrier()
  pltpu.sync_copy(accin_sh.at[pl.ds(sid * OROWS, OROWS)],
                  out_hbm.at[cid, 0, pl.ds(sid * OROWS, OROWS)])
  pltpu.sync_copy(accout_sh.at[pl.ds(sid * OROWS, OROWS)],
                  out_hbm.at[cid, 1, pl.ds(sid * OROWS, OROWS)])


_prop16 = _make_prop(16, 56, 7)
_prop8 = _make_prop(8, 56, 14)


NR = NPAD // 128
PR16 = NPAD * 16 // 128
PR8 = NPAD * 8 // 128


def _mask16(x):
  r = lax.broadcasted_iota(jnp.int32, x.shape, 0)
  l = lax.broadcasted_iota(jnp.int32, x.shape, 1)
  return jnp.where(8 * r + l // 16 < N, x, 0.0)


def _mask8(x):
  r = lax.broadcasted_iota(jnp.int32, x.shape, 0)
  l = lax.broadcasted_iota(jnp.int32, x.shape, 1)
  return jnp.where(16 * r + l // 8 < N, x, 0.0)


def _iota2(shape):
  a = lax.broadcasted_iota(jnp.int32, shape, 0)
  b = lax.broadcasted_iota(jnp.int32, shape, 1)
  return a, b


def _kron16(w):
  a, b = _iota2((128, 128))
  return jnp.where(a // 16 == b // 16, jnp.tile(w, (8, 8)), 0.0)


def _stage0(degp_ref, x1_ref, ndp_ref, nsp_ref, nd_ref):
  din = degp_ref[0, 0] + degp_ref[1, 0]
  dout = degp_ref[0, 1] + degp_ref[1, 1]
  p, l = _iota2((1024, 128))
  r8c0 = jnp.where((p // 8 == l) & (p % 8 == 0), 1.0, 0.0)
  def tonm(x):
    return jnp.dot(jnp.reshape(x, (NR, 1024)), r8c0,
                   preferred_element_type=jnp.float32)
  in_nm = tonm(din)
  out_nm = tonm(dout)
  ns = lax.rsqrt(jnp.maximum(out_nm, 1.0))
  nd = lax.rsqrt(jnp.maximum(in_nm, 1.0))
  l2, q = _iota2((128, 2048))
  e16 = jnp.where(q // 16 == l2, 1.0, 0.0)
  ndp_ref[...] = jnp.reshape(jnp.dot(nd, e16,
                                     preferred_element_type=jnp.float32),
                             (PR16, 128))
  nsp_ref[...] = jnp.reshape(jnp.dot(ns, e16,
                                     preferred_element_type=jnp.float32),
                             (PR16, 128))
  nd_ref[...] = nd
  r = lax.broadcasted_iota(jnp.int32, (NR, 128), 0)
  lm = lax.broadcasted_iota(jnp.int32, (NR, 128), 1)
  x1 = jnp.where(128 * r + lm < N, in_nm * ns, 0.0)
  l3, q8 = _iota2((128, 1024))
  e8c0 = jnp.where((q8 // 8 == l3) & (q8 % 8 == 0), 1.0, 0.0)
  x1_ref[...] = jnp.reshape(jnp.dot(x1, e8c0,
                                    preferred_element_type=jnp.float32),
                            (PR8, 128))


def _stage1(mp_ref, nd_ref, nsp_ref, w1_ref, b1_ref, x2_ref):
  m8 = mp_ref[0] + mp_ref[1]
  p, l = _iota2((1024, 128))
  r8c0 = jnp.where((p // 8 == l) & (p % 8 == 0), 1.0, 0.0)
  m_nm = jnp.dot(jnp.reshape(m8, (NR, 1024)), r8c0,
                 preferred_element_type=jnp.float32) * nd_ref[...]
  l2, q = _iota2((128, 2048))
  e16w1 = jnp.where(q // 16 == l2, 1.0, 0.0) * jnp.tile(w1_ref[...], (128, 128))
  h = jnp.reshape(jnp.dot(m_nm, e16w1, preferred_element_type=jnp.float32),
                  (PR16, 128))
  b1 = jnp.tile(b1_ref[...], (1, 8))
  x2_ref[...] = _mask16(jnp.maximum(h + b1, 0.0) * nsp_ref[...])


def _stage2(mp_ref, ndp_ref, nsp_ref, w2_ref, b2_ref, w3_ref, x3_ref):
  m = (mp_ref[0] + mp_ref[1]) * ndp_ref[...]
  b2 = jnp.tile(b2_ref[...], (1, 8))
  h = jnp.maximum(
      jnp.dot(m, _kron16(w2_ref[...]), preferred_element_type=jnp.float32)
      + b2, 0.0)
  x3_ref[...] = _mask16(
      jnp.dot(h, _kron16(w3_ref[...]), preferred_element_type=jnp.float32)
      * nsp_ref[...])


def _stage3(mp_ref, ndp_ref, nsp_ref, b3_ref, w4_ref, x4_ref):
  m = (mp_ref[0] + mp_ref[1]) * ndp_ref[...]
  b3 = jnp.tile(b3_ref[...], (1, 8))
  h = jnp.maximum(m + b3, 0.0)
  x4_ref[...] = _mask16(
      jnp.dot(h, _kron16(w4_ref[...]), preferred_element_type=jnp.float32)
      * nsp_ref[...])


def _stage4(mp_ref, ndp_ref, b4_ref, wc_ref, bc_ref, out_ref):
  m = (mp_ref[0] + mp_ref[1]) * ndp_ref[...]
  b4 = jnp.tile(b4_ref[...], (1, 8))
  h = _mask16(jnp.maximum(m + b4, 0.0))
  lane = jnp.sum(h, axis=0, keepdims=True)
  l, j = _iota2((128, 16))
  sel = jnp.where(l % 16 == j, 1.0, 0.0)
  g = jnp.dot(lane, sel, preferred_element_type=jnp.float32) * (1.0 / N)
  out_ref[...] = (
      jnp.dot(g[:, :4], wc_ref[...], preferred_element_type=jnp.float32)
      + bc_ref[...])


def _tc(body, out_shape, *args):
  return pl.pallas_call(body, out_shape=out_shape)(*args)


def _f32(*shape):
  return jax.ShapeDtypeStruct(shape, jnp.float32)


def kernel(edge_index, num_nodes, W1, b1, W2, b2, W3, b3, W4, b4, Wc, bc):
  del num_nodes
  pad = jnp.full((EPAD - E,), SENT, jnp.int32)
  src2 = jnp.concatenate([edge_index[0], pad]).reshape(EROWS, 128)
  dst2 = jnp.concatenate([edge_index[1], pad]).reshape(EROWS, 128)
  ones8 = jnp.ones((128, 8), jnp.float32)
  z8 = jnp.zeros((NPAD, 8), jnp.float32)
  z16 = jnp.zeros((NPAD, 16), jnp.float32)

  degp = _degrees(src2, dst2, ones8, z8)
  x1p, ndp, nsp, nd = _tc(
      _stage0,
      (_f32(PR8, 128), _f32(PR16, 128), _f32(PR16, 128), _f32(NR, 128)),
      degp.reshape(NC, 2, PR8, 128))
  m1 = _prop8(x1p.reshape(NPAD, 8), src2, dst2, z8)
  x2p = _tc(_stage1, _f32(PR16, 128), m1.reshape(NC, PR8, 128), nd, nsp,
            W1, b1.reshape(1, 16))
  m2 = _prop16(x2p.reshape(NPAD, 16), src2, dst2, z16)
  w3p = jnp.concatenate([W3, jnp.zeros((16, 8), jnp.float32)], axis=1)
  x3p = _tc(_stage2, _f32(PR16, 128), m2.reshape(NC, PR16, 128), ndp, nsp,
            W2, b2.reshape(1, 16), w3p)
  m3 = _prop16(x3p.reshape(NPAD, 16), src2, dst2, z16)
  b3p = jnp.concatenate([b3, jnp.zeros((8,), jnp.float32)]).reshape(1, 16)
  w4p = jnp.zeros((16, 16), jnp.float32).at[:8, :4].set(W4)
  x4p = _tc(_stage3, _f32(PR16, 128), m3.reshape(NC, PR16, 128), ndp, nsp,
            b3p, w4p)
  m4 = _prop16(x4p.reshape(NPAD, 16), src2, dst2, z16)
  b4p = jnp.concatenate([b4, jnp.zeros((12,), jnp.float32)]).reshape(1, 16)
  out = _tc(_stage4, _f32(1, 10), m4.reshape(NC, PR16, 128), ndp,
            b4p, Wc, bc.reshape(1, 10))
  return out
